# Initial kernel scaffold; baseline (speedup 1.0000x reference)
import jax
import jax.numpy as jnp
from jax.experimental import pallas as pl


def kernel(x, neighbors, lstm_Wi, lstm_Wh, lstm_bi, lstm_bh, W_self, b_self, W_neigh, b_neigh, Wg, bg, Wc, bc, Wf, bf, Ws, bs):
    raise NotImplementedError("write your pallas kernel here")


# speedup vs baseline: 2.3708x; 2.3708x over previous
import functools
import math

import jax
import jax.numpy as jnp
from jax import lax
from jax.experimental import pallas as pl
from jax.experimental.pallas import tpu as pltpu
from jax.experimental.pallas import tpu_sc as plsc

N = 10000
NP = 10240
DEG = 32
D = 128
HID = 64
NCOL = 3
CPAD = 16
NC = 2
NS = 16
NW = NC * NS
CH = 80
PER_W = NP * DEG // NW
NCHUNK = PER_W // CH
TN = 512
GRID = NP // TN
NEG = -1e30


def _sc_gather(table, idx2d, ncols):
    mesh = plsc.VectorSubcoreMesh(core_axis_name="c", subcore_axis_name="s")

    @functools.partial(
        pl.kernel,
        mesh=mesh,
        out_type=jax.ShapeDtypeStruct((NP * DEG, ncols), jnp.float32),
        compiler_params=pltpu.CompilerParams(use_tc_tiling_on_sc=False),
        scratch_types=[
            pltpu.VMEM((NCHUNK, CH), jnp.int32),
            pltpu.VMEM((CH, ncols), jnp.float32),
            pltpu.SemaphoreType.DMA,
        ],
    )
    def k(table_hbm, idx_hbm, out_hbm, idx_v, buf_v, sem):
        wid = lax.axis_index("s") * NC + lax.axis_index("c")
        pltpu.sync_copy(idx_hbm.at[pl.ds(wid * NCHUNK, NCHUNK)], idx_v)
        base = wid * PER_W

        def body(g, carry):
            pltpu.async_copy(table_hbm.at[idx_v.at[g]], buf_v, sem).wait()
            pltpu.sync_copy(buf_v, out_hbm.at[pl.ds(base + g * CH, CH)])
            return carry

        lax.fori_loop(0, NCHUNK, body, 0)

    return k(table, idx2d)


def _sc_hist(idx_flat):
    perh = N * DEG // NW
    mesh = plsc.VectorSubcoreMesh(core_axis_name="c", subcore_axis_name="s")

    @functools.partial(
        pl.kernel,
        mesh=mesh,
        out_type=jax.ShapeDtypeStruct((NW, NP), jnp.float32),
        compiler_params=pltpu.CompilerParams(needs_layout_passes=False),
        scratch_types=[
            pltpu.VMEM((perh,), jnp.int32),
            pltpu.VMEM((NP,), jnp.float32),
        ],
    )
    def k(idx_hbm, out_hbm, idx_v, bins_v):
        wid = lax.axis_index("s") * NC + lax.axis_index("c")
        pltpu.sync_copy(idx_hbm.at[pl.ds(wid * perh, perh)], idx_v)
        zeros16 = jnp.zeros((16,), jnp.float32)

        def zbody(i, c):
            bins_v[pl.ds(i * 16, 16)] = zeros16
            return c

        lax.fori_loop(0, NP // 16, zbody, 0)
        ones16 = jnp.ones((16,), jnp.float32)

        def body(i, c):
            v = idx_v[pl.ds(i * 16, 16)]
            plsc.addupdate_scatter(bins_v, [v], ones16)
            return c

        lax.fori_loop(0, perh // 16, body, 0)
        pltpu.sync_copy(bins_v, out_hbm.at[wid])

    return k(idx_flat)


def _tc_iter(nb, hself, gst, wiT_top, wiT_bot, whT, ws_top, ws_bot, wn,
             b4, b2, wg, first):
    scale = 1.0 / math.sqrt(D) if first else 1.0
    C = nb.shape[2]
    Cs = hself.shape[1]

    def body(nb_ref, hs_ref, gst_ref, wit_ref, wib_ref, wht_ref, wst_ref,
             wsb_ref, wn_ref, b4_ref, b2_ref, wg_ref,
             h2_ref, gout_ref, ms_ref, num_ref):
        i = pl.program_id(0)
        gstv = gst_ref[...]
        bias4 = b4_ref[...] + jnp.dot(gstv, wib_ref[...],
                                      preferred_element_type=jnp.float32)

        def step(t, carry):
            hs, c = carry
            xt = nb_ref[t] * scale
            gates = (jnp.dot(xt, wit_ref[...],
                             preferred_element_type=jnp.float32)
                     + jnp.dot(hs, wht_ref[...],
                               preferred_element_type=jnp.float32)
                     + bias4)
            ig = jax.nn.sigmoid(gates[:, 0:D])
            fg = jax.nn.sigmoid(gates[:, D:2 * D])
            gg = jnp.tanh(gates[:, 2 * D:3 * D])
            og = jax.nn.sigmoid(gates[:, 3 * D:4 * D])
            c = fg * c + ig * gg
            hs = og * jnp.tanh(c)
            return (hs, c)

        z = jnp.zeros((TN, D), jnp.float32)
        hs, _ = lax.fori_loop(0, DEG, step, (z, z))

        h2 = (jnp.dot(hs_ref[...] * scale, wst_ref[...],
                      preferred_element_type=jnp.float32)
              + jnp.dot(gstv, wsb_ref[...],
                        preferred_element_type=jnp.float32)
              + jnp.dot(hs, wn_ref[...],
                        preferred_element_type=jnp.float32)
              + b2_ref[...])
        h2 = jnp.where(h2 > 0, h2, 0.01 * h2)
        h2_ref[...] = h2

        logit = jnp.dot(h2, wg_ref[...], preferred_element_type=jnp.float32)
        rows = lax.broadcasted_iota(jnp.int32, (TN, 1), 0) + i * TN
        lg = jnp.where(rows < N, logit, NEG)

        @pl.when(i == 0)
        def _():
            ms_ref[0] = NEG
            ms_ref[1] = 0.0
            num_ref[...] = jnp.zeros_like(num_ref)

        m_old = ms_ref[0]
        m_new = jnp.maximum(m_old, jnp.max(lg))
        sc = jnp.exp(m_old - m_new)
        e = jnp.where(rows < N, jnp.exp(lg - m_new), 0.0)
        s_new = ms_ref[1] * sc + jnp.sum(e)
        num_new = num_ref[...] * sc + lax.dot_general(
            e, h2, (((0,), (0,)), ((), ())),
            preferred_element_type=jnp.float32)
        ms_ref[0] = m_new
        ms_ref[1] = s_new
        num_ref[...] = num_new
        gout_ref[...] = num_new / s_new

    return pl.pallas_call(
        body,
        grid=(GRID,),
        in_specs=[
            pl.BlockSpec((DEG, TN, C), lambda i: (0, i, 0)),
            pl.BlockSpec((TN, Cs), lambda i: (i, 0)),
            pl.BlockSpec((1, HID), lambda i: (0, 0)),
            pl.BlockSpec((C, 4 * D), lambda i: (0, 0)),
            pl.BlockSpec((HID, 4 * D), lambda i: (0, 0)),
            pl.BlockSpec((D, 4 * D), lambda i: (0, 0)),
            pl.BlockSpec((Cs, HID), lambda i: (0, 0)),
            pl.BlockSpec((HID, HID), lambda i: (0, 0)),
            pl.BlockSpec((D, HID), lambda i: (0, 0)),
            pl.BlockSpec((1, 4 * D), lambda i: (0, 0)),
            pl.BlockSpec((1, HID), lambda i: (0, 0)),
            pl.BlockSpec((HID, 1), lambda i: (0, 0)),
        ],
        out_specs=[
            pl.BlockSpec((TN, HID), lambda i: (i, 0)),
            pl.BlockSpec((1, HID), lambda i: (0, 0)),
        ],
        out_shape=[
            jax.ShapeDtypeStruct((NP, HID), jnp.float32),
            jax.ShapeDtypeStruct((1, HID), jnp.float32),
        ],
        scratch_shapes=[
            pltpu.SMEM((2,), jnp.float32),
            pltpu.VMEM((1, HID), jnp.float32),
        ],
    )(nb, hself, gst, wiT_top, wiT_bot, whT, ws_top, ws_bot, wn, b4, b2, wg)


def _tc_feat(h2, gst, hist, wc_top, wc_bot):
    def body(h2_ref, gst_ref, hist_ref, wt_ref, wb_ref, out_ref):
        ones = jnp.ones((NW, 1), jnp.float32)
        cnt = lax.dot_general(hist_ref[...], ones, (((0,), (0,)), ((), ())),
                              preferred_element_type=jnp.float32)
        feat = (jnp.dot(h2_ref[...], wt_ref[...],
                        preferred_element_type=jnp.float32)
                + jnp.dot(gst_ref[...], wb_ref[...],
                          preferred_element_type=jnp.float32))
        out_ref[...] = feat * lax.rsqrt(jnp.maximum(cnt, 1.0))

    return pl.pallas_call(
        body,
        grid=(GRID,),
        in_specs=[
            pl.BlockSpec((TN, HID), lambda i: (i, 0)),
            pl.BlockSpec((1, HID), lambda i: (0, 0)),
            pl.BlockSpec((NW, TN), lambda i: (0, i)),
            pl.BlockSpec((HID, CPAD), lambda i: (0, 0)),
            pl.BlockSpec((HID, CPAD), lambda i: (0, 0)),
        ],
        out_specs=pl.BlockSpec((TN, CPAD), lambda i: (i, 0)),
        out_shape=jax.ShapeDtypeStruct((NP, CPAD), jnp.float32),
    )(h2, gst, hist, wc_top, wc_bot)


def _tc_final(nbf, bc_pad, wf_pad, ws_pad, bs):
    def body(nb_ref, bc_ref, wf_ref, ws_ref, bs_ref,
             col_ref, sat_ref, ms_ref, num_ref):
        i = pl.program_id(0)
        agg = jnp.sum(nb_ref[...], axis=0)
        g2 = agg * (1.0 / math.sqrt(DEG)) + bc_ref[...]
        cols = lax.broadcasted_iota(jnp.int32, (TN, CPAD), 1)
        cvalid = cols < NCOL
        g2m = jnp.where(cvalid, g2, NEG)
        rmax = jnp.max(g2m, axis=1, keepdims=True)
        ex = jnp.where(cvalid, jnp.exp(g2m - rmax), 0.0)
        colors = ex / jnp.sum(ex, axis=1, keepdims=True)
        col_ref[...] = colors

        logit = jnp.dot(colors, wf_ref[...],
                        preferred_element_type=jnp.float32)
        rows = lax.broadcasted_iota(jnp.int32, (TN, 1), 0) + i * TN
        lg = jnp.where(rows < N, logit, NEG)

        @pl.when(i == 0)
        def _():
            ms_ref[0] = NEG
            ms_ref[1] = 0.0
            num_ref[...] = jnp.zeros_like(num_ref)

        m_old = ms_ref[0]
        m_new = jnp.maximum(m_old, jnp.max(lg))
        sc = jnp.exp(m_old - m_new)
        e = jnp.where(rows < N, jnp.exp(lg - m_new), 0.0)
        s_new = ms_ref[1] * sc + jnp.sum(e)
        num_new = num_ref[...] * sc + lax.dot_general(
            e, colors, (((0,), (0,)), ((), ())),
            preferred_element_type=jnp.float32)
        ms_ref[0] = m_new
        ms_ref[1] = s_new
        num_ref[...] = num_new
        pooled = num_new / s_new
        sat_ref[...] = jax.nn.sigmoid(
            jnp.dot(pooled, ws_ref[...], preferred_element_type=jnp.float32)
            + bs_ref[...])

    return pl.pallas_call(
        body,
        grid=(GRID,),
        in_specs=[
            pl.BlockSpec((DEG, TN, CPAD), lambda i: (0, i, 0)),
            pl.BlockSpec((1, CPAD), lambda i: (0, 0)),
            pl.BlockSpec((CPAD, 1), lambda i: (0, 0)),
            pl.BlockSpec((CPAD, 1), lambda i: (0, 0)),
            pl.BlockSpec((1, 1), lambda i: (0, 0)),
        ],
        out_specs=[
            pl.BlockSpec((TN, CPAD), lambda i: (i, 0)),
            pl.BlockSpec((1, 1), lambda i: (0, 0)),
        ],
        out_shape=[
            jax.ShapeDtypeStruct((NP, CPAD), jnp.float32),
            jax.ShapeDtypeStruct((1, 1), jnp.float32),
        ],
        scratch_shapes=[
            pltpu.SMEM((2,), jnp.float32),
            pltpu.VMEM((1, CPAD), jnp.float32),
        ],
    )(nbf, bc_pad, wf_pad, ws_pad, bs)


def kernel(x, neighbors, lstm_Wi, lstm_Wh, lstm_bi, lstm_bh, W_self, b_self,
           W_neigh, b_neigh, Wg, bg, Wc, bc, Wf, bf, Ws, bs):
    x_pad = jnp.pad(x, ((0, NP - N), (0, 0)))
    nbr_pad = jnp.pad(neighbors, ((0, NP - N), (0, 0)))
    idxT = jnp.transpose(nbr_pad).reshape(NW * NCHUNK, CH)
    wiT = jnp.transpose(lstm_Wi)
    whT = jnp.transpose(lstm_Wh)
    wiT_bot = wiT[HID:]
    ws_bot = W_self[HID:]
    b4 = (lstm_bi + lstm_bh).reshape(1, 4 * D)
    b2 = (b_self + b_neigh).reshape(1, HID)
    wc_pad = jnp.pad(Wc, ((0, 0), (0, CPAD - NCOL)))
    bc_pad = jnp.pad(bc, (0, CPAD - NCOL)).reshape(1, CPAD)
    wf_pad = jnp.pad(Wf, ((0, CPAD - NCOL), (0, 0)))
    ws_pad = jnp.pad(Ws, ((0, CPAD - NCOL), (0, 0)))
    zgst = jnp.zeros((1, HID), jnp.float32)
    zbot = jnp.zeros((HID, 4 * D), jnp.float32)
    zsb = jnp.zeros((HID, HID), jnp.float32)

    hist = _sc_hist(neighbors.reshape(-1))

    nb1 = _sc_gather(x_pad, idxT, D).reshape(DEG, NP, D)
    h2, gst = _tc_iter(nb1, x_pad, zgst, wiT, zbot, whT, W_self, zsb,
                       W_neigh, b4, b2, Wg, first=True)

    for _ in range(2):
        nb = _sc_gather(h2, idxT, HID).reshape(DEG, NP, HID)
        h2, gst = _tc_iter(nb, h2, gst, wiT[:HID], wiT_bot, whT,
                           W_self[:HID], ws_bot, W_neigh, b4, b2, Wg,
                           first=False)

    scaled = _tc_feat(h2, gst, hist, wc_pad[:HID], wc_pad[HID:])
    nbf = _sc_gather(scaled, idxT, CPAD).reshape(DEG, NP, CPAD)
    colors_pad, sat = _tc_final(nbf, bc_pad, wf_pad, ws_pad,
                                bs.reshape(1, 1))

    return colors_pad[:N, :NCOL], sat.reshape(())


# speedup vs baseline: 2.6450x; 1.1157x over previous
import functools
import math

import jax
import jax.numpy as jnp
from jax import lax
from jax.experimental import pallas as pl
from jax.experimental.pallas import tpu as pltpu
from jax.experimental.pallas import tpu_sc as plsc

N = 10000
NP = 10240
DEG = 32
D = 128
HID = 64
NCOL = 3
CPAD = 16
NC = 2
NS = 16
NW = NC * NS
CH = 128
PER_W = NP * DEG // NW
NCHUNK = PER_W // CH
NBUF = 4
TN = 512
GRID = NP // TN
NEG = -1e30


def _sc_gather(table, idx2d, ncols):
    mesh = plsc.VectorSubcoreMesh(core_axis_name="c", subcore_axis_name="s")

    @functools.partial(
        pl.kernel,
        mesh=mesh,
        out_type=jax.ShapeDtypeStruct((NP * DEG, ncols), jnp.float32),
        compiler_params=pltpu.CompilerParams(use_tc_tiling_on_sc=False),
        scratch_types=(
            [pltpu.VMEM((NCHUNK, CH), jnp.int32)]
            + [pltpu.VMEM((CH, ncols), jnp.float32) for _ in range(NBUF)]
            + [pltpu.SemaphoreType.DMA for _ in range(2 * NBUF)]
        ),
    )
    def k(table_hbm, idx_hbm, out_hbm, idx_v, *rest):
        bufs = rest[:NBUF]
        gsem = rest[NBUF:2 * NBUF]
        wsem = rest[2 * NBUF:]
        wid = lax.axis_index("s") * NC + lax.axis_index("c")
        pltpu.sync_copy(idx_hbm.at[pl.ds(wid * NCHUNK, NCHUNK)], idx_v)
        base = wid * PER_W

        for b in range(NBUF):
            pltpu.async_copy(table_hbm.at[idx_v.at[b]], bufs[b], gsem[b])

        def body(q, carry):
            g0 = q * NBUF
            for b in range(NBUF):
                out_slc = out_hbm.at[pl.ds(base + (g0 + b) * CH, CH)]
                pltpu.make_async_copy(
                    table_hbm.at[idx_v.at[g0 + b]], bufs[b], gsem[b]).wait()
                pltpu.async_copy(bufs[b], out_slc, wsem[b])

            @pl.when(q < NCHUNK // NBUF - 1)
            def _():
                for b in range(NBUF):
                    out_slc = out_hbm.at[pl.ds(base + (g0 + b) * CH, CH)]
                    pltpu.make_async_copy(bufs[b], out_slc, wsem[b]).wait()
                    pltpu.async_copy(
                        table_hbm.at[idx_v.at[g0 + NBUF + b]], bufs[b],
                        gsem[b])
            return carry

        lax.fori_loop(0, NCHUNK // NBUF, body, 0)
        for b in range(NBUF):
            out_slc = out_hbm.at[pl.ds(base + PER_W - (NBUF - b) * CH, CH)]
            pltpu.make_async_copy(bufs[b], out_slc, wsem[b]).wait()

    return k(table, idx2d)


def _sc_hist(idx_flat):
    perh = N * DEG // NW
    mesh = plsc.VectorSubcoreMesh(core_axis_name="c", subcore_axis_name="s")

    @functools.partial(
        pl.kernel,
        mesh=mesh,
        out_type=jax.ShapeDtypeStruct((NW, NP), jnp.float32),
        compiler_params=pltpu.CompilerParams(needs_layout_passes=False),
        scratch_types=[
            pltpu.VMEM((perh,), jnp.int32),
            pltpu.VMEM((NP,), jnp.float32),
        ],
    )
    def k(idx_hbm, out_hbm, idx_v, bins_v):
        wid = lax.axis_index("s") * NC + lax.axis_index("c")
        pltpu.sync_copy(idx_hbm.at[pl.ds(wid * perh, perh)], idx_v)
        zeros16 = jnp.zeros((16,), jnp.float32)

        def zbody(i, c):
            bins_v[pl.ds(i * 16, 16)] = zeros16
            return c

        lax.fori_loop(0, NP // 16, zbody, 0)
        ones16 = jnp.ones((16,), jnp.float32)

        def body(i, c):
            v = idx_v[pl.ds(i * 16, 16)]
            plsc.addupdate_scatter(bins_v, [v], ones16)
            return c

        lax.fori_loop(0, perh // 16, body, 0)
        pltpu.sync_copy(bins_v, out_hbm.at[wid])

    return k(idx_flat)


def _tc_iter(nb, hself, gst, wiT_top, wiT_bot, whT, ws_top, ws_bot, wn,
             b4, b2, wg, first):
    scale = 1.0 / math.sqrt(D) if first else 1.0
    C = nb.shape[2]
    Cs = hself.shape[1]

    def body(nb_ref, hs_ref, gst_ref, wit_ref, wib_ref, wht_ref, wst_ref,
             wsb_ref, wn_ref, b4_ref, b2_ref, wg_ref,
             h2_ref, gout_ref, ms_ref, num_ref):
        i = pl.program_id(0)
        gstv = gst_ref[...]
        bias4 = b4_ref[...] + jnp.dot(gstv, wib_ref[...],
                                      preferred_element_type=jnp.float32)
        wit = wit_ref[...].astype(jnp.bfloat16)
        wht = wht_ref[...].astype(jnp.bfloat16)

        def step(t, carry):
            hs, c = carry
            xt = (nb_ref[t] * scale).astype(jnp.bfloat16)
            gates = (jnp.dot(xt, wit,
                             preferred_element_type=jnp.float32)
                     + jnp.dot(hs.astype(jnp.bfloat16), wht,
                               preferred_element_type=jnp.float32)
                     + bias4)
            ig = jax.nn.sigmoid(gates[:, 0:D])
            fg = jax.nn.sigmoid(gates[:, D:2 * D])
            gg = jnp.tanh(gates[:, 2 * D:3 * D])
            og = jax.nn.sigmoid(gates[:, 3 * D:4 * D])
            c = fg * c + ig * gg
            hs = og * jnp.tanh(c)
            return (hs, c)

        z = jnp.zeros((TN, D), jnp.float32)
        hs, _ = lax.fori_loop(0, DEG, step, (z, z))

        h2 = (jnp.dot(hs_ref[...] * scale, wst_ref[...],
                      preferred_element_type=jnp.float32)
              + jnp.dot(gstv, wsb_ref[...],
                        preferred_element_type=jnp.float32)
              + jnp.dot(hs, wn_ref[...],
                        preferred_element_type=jnp.float32)
              + b2_ref[...])
        h2 = jnp.where(h2 > 0, h2, 0.01 * h2)
        h2_ref[...] = h2

        logit = jnp.dot(h2, wg_ref[...], preferred_element_type=jnp.float32)
        rows = lax.broadcasted_iota(jnp.int32, (TN, 1), 0) + i * TN
        lg = jnp.where(rows < N, logit, NEG)

        @pl.when(i == 0)
        def _():
            ms_ref[0] = NEG
            ms_ref[1] = 0.0
            num_ref[...] = jnp.zeros_like(num_ref)

        m_old = ms_ref[0]
        m_new = jnp.maximum(m_old, jnp.max(lg))
        sc = jnp.exp(m_old - m_new)
        e = jnp.where(rows < N, jnp.exp(lg - m_new), 0.0)
        s_new = ms_ref[1] * sc + jnp.sum(e)
        num_new = num_ref[...] * sc + lax.dot_general(
            e, h2, (((0,), (0,)), ((), ())),
            preferred_element_type=jnp.float32)
        ms_ref[0] = m_new
        ms_ref[1] = s_new
        num_ref[...] = num_new
        gout_ref[...] = num_new / s_new

    return pl.pallas_call(
        body,
        grid=(GRID,),
        in_specs=[
            pl.BlockSpec((DEG, TN, C), lambda i: (0, i, 0)),
            pl.BlockSpec((TN, Cs), lambda i: (i, 0)),
            pl.BlockSpec((1, HID), lambda i: (0, 0)),
            pl.BlockSpec((C, 4 * D), lambda i: (0, 0)),
            pl.BlockSpec((HID, 4 * D), lambda i: (0, 0)),
            pl.BlockSpec((D, 4 * D), lambda i: (0, 0)),
            pl.BlockSpec((Cs, HID), lambda i: (0, 0)),
            pl.BlockSpec((HID, HID), lambda i: (0, 0)),
            pl.BlockSpec((D, HID), lambda i: (0, 0)),
            pl.BlockSpec((1, 4 * D), lambda i: (0, 0)),
            pl.BlockSpec((1, HID), lambda i: (0, 0)),
            pl.BlockSpec((HID, 1), lambda i: (0, 0)),
        ],
        out_specs=[
            pl.BlockSpec((TN, HID), lambda i: (i, 0)),
            pl.BlockSpec((1, HID), lambda i: (0, 0)),
        ],
        out_shape=[
            jax.ShapeDtypeStruct((NP, HID), jnp.float32),
            jax.ShapeDtypeStruct((1, HID), jnp.float32),
        ],
        scratch_shapes=[
            pltpu.SMEM((2,), jnp.float32),
            pltpu.VMEM((1, HID), jnp.float32),
        ],
    )(nb, hself, gst, wiT_top, wiT_bot, whT, ws_top, ws_bot, wn, b4, b2, wg)


def _tc_feat(h2, gst, hist, wc_top, wc_bot):
    def body(h2_ref, gst_ref, hist_ref, wt_ref, wb_ref, out_ref):
        ones = jnp.ones((NW, 1), jnp.float32)
        cnt = lax.dot_general(hist_ref[...], ones, (((0,), (0,)), ((), ())),
                              preferred_element_type=jnp.float32)
        feat = (jnp.dot(h2_ref[...], wt_ref[...],
                        preferred_element_type=jnp.float32)
                + jnp.dot(gst_ref[...], wb_ref[...],
                          preferred_element_type=jnp.float32))
        out_ref[...] = feat * lax.rsqrt(jnp.maximum(cnt, 1.0))

    return pl.pallas_call(
        body,
        grid=(GRID,),
        in_specs=[
            pl.BlockSpec((TN, HID), lambda i: (i, 0)),
            pl.BlockSpec((1, HID), lambda i: (0, 0)),
            pl.BlockSpec((NW, TN), lambda i: (0, i)),
            pl.BlockSpec((HID, CPAD), lambda i: (0, 0)),
            pl.BlockSpec((HID, CPAD), lambda i: (0, 0)),
        ],
        out_specs=pl.BlockSpec((TN, CPAD), lambda i: (i, 0)),
        out_shape=jax.ShapeDtypeStruct((NP, CPAD), jnp.float32),
    )(h2, gst, hist, wc_top, wc_bot)


def _tc_final(nbf, bc_pad, wf_pad, ws_pad, bs):
    def body(nb_ref, bc_ref, wf_ref, ws_ref, bs_ref,
             col_ref, sat_ref, ms_ref, num_ref):
        i = pl.program_id(0)
        agg = jnp.sum(nb_ref[...], axis=0)
        g2 = agg * (1.0 / math.sqrt(DEG)) + bc_ref[...]
        cols = lax.broadcasted_iota(jnp.int32, (TN, CPAD), 1)
        cvalid = cols < NCOL
        g2m = jnp.where(cvalid, g2, NEG)
        rmax = jnp.max(g2m, axis=1, keepdims=True)
        ex = jnp.where(cvalid, jnp.exp(g2m - rmax), 0.0)
        colors = ex / jnp.sum(ex, axis=1, keepdims=True)
        col_ref[...] = colors

        logit = jnp.dot(colors, wf_ref[...],
                        preferred_element_type=jnp.float32)
        rows = lax.broadcasted_iota(jnp.int32, (TN, 1), 0) + i * TN
        lg = jnp.where(rows < N, logit, NEG)

        @pl.when(i == 0)
        def _():
            ms_ref[0] = NEG
            ms_ref[1] = 0.0
            num_ref[...] = jnp.zeros_like(num_ref)

        m_old = ms_ref[0]
        m_new = jnp.maximum(m_old, jnp.max(lg))
        sc = jnp.exp(m_old - m_new)
        e = jnp.where(rows < N, jnp.exp(lg - m_new), 0.0)
        s_new = ms_ref[1] * sc + jnp.sum(e)
        num_new = num_ref[...] * sc + lax.dot_general(
            e, colors, (((0,), (0,)), ((), ())),
            preferred_element_type=jnp.float32)
        ms_ref[0] = m_new
        ms_ref[1] = s_new
        num_ref[...] = num_new
        pooled = num_new / s_new
        sat_ref[...] = jax.nn.sigmoid(
            jnp.dot(pooled, ws_ref[...], preferred_element_type=jnp.float32)
            + bs_ref[...])

    return pl.pallas_call(
        body,
        grid=(GRID,),
        in_specs=[
            pl.BlockSpec((DEG, TN, CPAD), lambda i: (0, i, 0)),
            pl.BlockSpec((1, CPAD), lambda i: (0, 0)),
            pl.BlockSpec((CPAD, 1), lambda i: (0, 0)),
            pl.BlockSpec((CPAD, 1), lambda i: (0, 0)),
            pl.BlockSpec((1, 1), lambda i: (0, 0)),
        ],
        out_specs=[
            pl.BlockSpec((TN, CPAD), lambda i: (i, 0)),
            pl.BlockSpec((1, 1), lambda i: (0, 0)),
        ],
        out_shape=[
            jax.ShapeDtypeStruct((NP, CPAD), jnp.float32),
            jax.ShapeDtypeStruct((1, 1), jnp.float32),
        ],
        scratch_shapes=[
            pltpu.SMEM((2,), jnp.float32),
            pltpu.VMEM((1, CPAD), jnp.float32),
        ],
    )(nbf, bc_pad, wf_pad, ws_pad, bs)


def kernel(x, neighbors, lstm_Wi, lstm_Wh, lstm_bi, lstm_bh, W_self, b_self,
           W_neigh, b_neigh, Wg, bg, Wc, bc, Wf, bf, Ws, bs):
    x_pad = jnp.pad(x, ((0, NP - N), (0, 0)))
    nbr_pad = jnp.pad(neighbors, ((0, NP - N), (0, 0)))
    idxT = jnp.transpose(nbr_pad).reshape(NW * NCHUNK, CH)
    wiT = jnp.transpose(lstm_Wi)
    whT = jnp.transpose(lstm_Wh)
    wiT_bot = wiT[HID:]
    ws_bot = W_self[HID:]
    b4 = (lstm_bi + lstm_bh).reshape(1, 4 * D)
    b2 = (b_self + b_neigh).reshape(1, HID)
    wc_pad = jnp.pad(Wc, ((0, 0), (0, CPAD - NCOL)))
    bc_pad = jnp.pad(bc, (0, CPAD - NCOL)).reshape(1, CPAD)
    wf_pad = jnp.pad(Wf, ((0, CPAD - NCOL), (0, 0)))
    ws_pad = jnp.pad(Ws, ((0, CPAD - NCOL), (0, 0)))
    zgst = jnp.zeros((1, HID), jnp.float32)
    zbot = jnp.zeros((HID, 4 * D), jnp.float32)
    zsb = jnp.zeros((HID, HID), jnp.float32)

    hist = _sc_hist(neighbors.reshape(-1))

    nb1 = _sc_gather(x_pad, idxT, D).reshape(DEG, NP, D)
    h2, gst = _tc_iter(nb1, x_pad, zgst, wiT, zbot, whT, W_self, zsb,
                       W_neigh, b4, b2, Wg, first=True)

    for _ in range(2):
        nb = _sc_gather(h2, idxT, HID).reshape(DEG, NP, HID)
        h2, gst = _tc_iter(nb, h2, gst, wiT[:HID], wiT_bot, whT,
                           W_self[:HID], ws_bot, W_neigh, b4, b2, Wg,
                           first=False)

    scaled = _tc_feat(h2, gst, hist, wc_pad[:HID], wc_pad[HID:])
    nbf = _sc_gather(scaled, idxT, CPAD).reshape(DEG, NP, CPAD)
    colors_pad, sat = _tc_final(nbf, bc_pad, wf_pad, ws_pad,
                                bs.reshape(1, 1))

    return colors_pad[:N, :NCOL], sat.reshape(())
